# baseline (speedup 1.0000x reference)
import functools

import jax
import jax.numpy as jnp
from jax import lax
from jax.experimental import pallas as pl
from jax.experimental.pallas import tpu as pltpu
from jax.experimental.pallas import tpu_sc as plsc

_CAPACITY = 65536
_EMBED = 256
_PTR = 0
_NFEAT = 4096

_NTILE = 16
_TROWS = _NFEAT // _NTILE
_LANE = 16
_NSLC = _EMBED // _LANE

_NCH = 4
_CROWS = _CAPACITY // _NCH
_NBUF = 3
_SLACK = 1


def _sc_mean_body(f_hbm, row_hbm, fbuf, acc, partials, gath):
    core = lax.axis_index("c")
    tile = lax.axis_index("s")

    @pl.when(core == 0)
    def _reduce():
        base = tile * _TROWS
        pltpu.sync_copy(f_hbm.at[pl.ds(base, _TROWS), :], fbuf)
        for c in range(_NSLC):
            sl = pl.ds(c * _LANE, _LANE)

            def body(r, v):
                return v + fbuf[r, sl]

            acc[0, sl] = lax.fori_loop(0, _TROWS, body,
                                       jnp.zeros((_LANE,), jnp.float32))
        pltpu.sync_copy(acc, partials.at[pl.ds(tile, 1), :])

    plsc.subcore_barrier()

    @pl.when((core == 0) & (tile == 0))
    def _combine():
        pltpu.sync_copy(partials, gath)
        for c in range(_NSLC):
            sl = pl.ds(c * _LANE, _LANE)
            v = jnp.zeros((_LANE,), jnp.float32)
            for t in range(_NTILE):
                v = v + gath[t, sl]
            acc[0, sl] = v * (1.0 / _NFEAT)
        pltpu.sync_copy(acc, row_hbm)


@functools.partial(
    pl.kernel,
    out_type=jax.ShapeDtypeStruct((1, _EMBED), jnp.float32),
    mesh=plsc.VectorSubcoreMesh(core_axis_name="c", subcore_axis_name="s"),
    scratch_types=[
        pltpu.VMEM((_TROWS, _EMBED), jnp.float32),
        pltpu.VMEM((1, _EMBED), jnp.float32),
        pltpu.VMEM_SHARED((_NTILE, _EMBED), jnp.float32),
        pltpu.VMEM((_NTILE, _EMBED), jnp.float32),
    ],
)
def _sc_mean(f_hbm, row_hbm, fbuf, acc, partials, gath):
    _sc_mean_body(f_hbm, row_hbm, fbuf, acc, partials, gath)


def _tc_body(row_hbm, m_hbm, o_hbm, bufs, rowbuf,
             in_sems, out_sems, row_in_sem, row_sem):
    def in_copy(i):
        return pltpu.make_async_copy(
            m_hbm.at[pl.ds(i * _CROWS, _CROWS), :],
            bufs.at[i % _NBUF],
            in_sems.at[i % _NBUF],
        )

    def out_copy(i):
        return pltpu.make_async_copy(
            bufs.at[i % _NBUF],
            o_hbm.at[pl.ds(i * _CROWS, _CROWS), :],
            out_sems.at[i % _NBUF],
        )

    rcopy = pltpu.make_async_copy(row_hbm, rowbuf, row_in_sem)
    rcopy.start()
    for b in range(_NBUF):
        in_copy(b).start()
    rcopy.wait()

    patch = pltpu.make_async_copy(rowbuf, o_hbm.at[pl.ds(_PTR, 1), :], row_sem)
    for i in range(_NCH):
        in_copy(i).wait()
        out_copy(i).start()
        j = i - _SLACK
        if j >= 0:
            out_copy(j).wait()
            if j + _NBUF < _NCH:
                in_copy(j + _NBUF).start()
            if j == _PTR // _CROWS:
                patch.start()
    for j in range(max(0, _NCH - _SLACK), _NCH):
        out_copy(j).wait()
    patch.wait()


def kernel(feature, memory):
    row = _sc_mean(feature)
    return pl.pallas_call(
        _tc_body,
        in_specs=[
            pl.BlockSpec(memory_space=pl.ANY),
            pl.BlockSpec(memory_space=pl.ANY),
        ],
        out_specs=pl.BlockSpec(memory_space=pl.ANY),
        out_shape=jax.ShapeDtypeStruct((_CAPACITY, _EMBED), jnp.float32),
        scratch_shapes=[
            pltpu.VMEM((_NBUF, _CROWS, _EMBED), jnp.float32),
            pltpu.VMEM((1, _EMBED), jnp.float32),
            pltpu.SemaphoreType.DMA((_NBUF,)),
            pltpu.SemaphoreType.DMA((_NBUF,)),
            pltpu.SemaphoreType.DMA,
            pltpu.SemaphoreType.DMA,
        ],
    )(row, memory)


# speedup vs baseline: 1.9293x; 1.9293x over previous
import jax
import jax.numpy as jnp
from jax.experimental import pallas as pl
from jax.experimental.pallas import tpu as pltpu

_CAPACITY = 65536
_EMBED = 256
_PTR = 0
_NFEAT = 4096

_NCH = 4
_CROWS = _CAPACITY // _NCH
_NBUF = 3
_SLACK = 1


def _body(f_hbm, m_hbm, o_hbm, fvmem, bufs, rowbuf,
          in_sems, out_sems, f_sem, row_sem):
    def in_copy(i):
        return pltpu.make_async_copy(
            m_hbm.at[pl.ds(i * _CROWS, _CROWS), :],
            bufs.at[i % _NBUF],
            in_sems.at[i % _NBUF],
        )

    def out_copy(i):
        return pltpu.make_async_copy(
            bufs.at[i % _NBUF],
            o_hbm.at[pl.ds(i * _CROWS, _CROWS), :],
            out_sems.at[i % _NBUF],
        )

    fcopy = pltpu.make_async_copy(f_hbm, fvmem, f_sem)
    fcopy.start()
    for b in range(_NBUF):
        in_copy(b).start()
    fcopy.wait()
    rowbuf[...] = jnp.sum(fvmem[...], axis=0, keepdims=True) * (1.0 / _NFEAT)

    patch = pltpu.make_async_copy(rowbuf, o_hbm.at[pl.ds(_PTR, 1), :], row_sem)
    for i in range(_NCH):
        in_copy(i).wait()
        out_copy(i).start()
        j = i - _SLACK
        if j >= 0:
            out_copy(j).wait()
            if j + _NBUF < _NCH:
                in_copy(j + _NBUF).start()
            if j == _PTR // _CROWS:
                patch.start()
    for j in range(max(0, _NCH - _SLACK), _NCH):
        out_copy(j).wait()
    patch.wait()


def kernel(feature, memory):
    return pl.pallas_call(
        _body,
        in_specs=[
            pl.BlockSpec(memory_space=pl.ANY),
            pl.BlockSpec(memory_space=pl.ANY),
        ],
        out_specs=pl.BlockSpec(memory_space=pl.ANY),
        out_shape=jax.ShapeDtypeStruct((_CAPACITY, _EMBED), jnp.float32),
        scratch_shapes=[
            pltpu.VMEM((_NFEAT, _EMBED), jnp.float32),
            pltpu.VMEM((_NBUF, _CROWS, _EMBED), jnp.float32),
            pltpu.VMEM((1, _EMBED), jnp.float32),
            pltpu.SemaphoreType.DMA((_NBUF,)),
            pltpu.SemaphoreType.DMA((_NBUF,)),
            pltpu.SemaphoreType.DMA,
            pltpu.SemaphoreType.DMA,
        ],
    )(feature, memory)
